# baseline (speedup 1.0000x reference)
import functools

import jax
import jax.numpy as jnp
from jax import lax
from jax.experimental import pallas as pl
from jax.experimental.pallas import tpu as pltpu
from jax.experimental.pallas import tpu_sc as plsc

N = 10000
E = 320000
HEADS = 8
CH = 32
D = HEADS * CH

NC = 2
NS = 16

NPAD = 10112
DUMMY = N
EE = E + N
B = 64
CPT = 324
EP = NS * CPT * B
ZSLAB = NPAD // NS

_f32 = jnp.float32


_BM = 400


def _tc1_body(x_r, w1_r, b1_r, wc1_r, was_r, wad_r, g_r, as_r, ad_r, m_r):
    h1 = jnp.maximum(
        jnp.dot(x_r[...], w1_r[...], preferred_element_type=_f32) + b1_r[...], 0.0)
    g = jnp.dot(h1, wc1_r[...], preferred_element_type=_f32)
    a_s = jnp.dot(g, was_r[...], preferred_element_type=_f32,
                  precision=lax.Precision.HIGHEST)
    a_d = jnp.dot(g, wad_r[...], preferred_element_type=_f32,
                  precision=lax.Precision.HIGHEST)
    g_r[...] = g
    as_r[...] = a_s
    ad_r[...] = a_d

    @pl.when(pl.program_id(0) == 0)
    def _():
        m_r[...] = jnp.full((1, 16), -jnp.inf, _f32)

    m_r[...] = jnp.maximum(m_r[...], jnp.max(a_s, axis=0, keepdims=True))


def _tc1(x, w1, b1r, wc1, was, wad):
    grid = N // _BM
    return pl.pallas_call(
        _tc1_body,
        grid=(grid,),
        in_specs=[
            pl.BlockSpec((_BM, 128), lambda i: (i, 0)),
            pl.BlockSpec((128, D), lambda i: (0, 0)),
            pl.BlockSpec((1, D), lambda i: (0, 0)),
            pl.BlockSpec((D, D), lambda i: (0, 0)),
            pl.BlockSpec((D, 16), lambda i: (0, 0)),
            pl.BlockSpec((D, 16), lambda i: (0, 0)),
        ],
        out_specs=[
            pl.BlockSpec((_BM, D), lambda i: (i, 0)),
            pl.BlockSpec((_BM, 16), lambda i: (i, 0)),
            pl.BlockSpec((_BM, 16), lambda i: (i, 0)),
            pl.BlockSpec((1, 16), lambda i: (0, 0)),
        ],
        out_shape=[
            jax.ShapeDtypeStruct((N, D), _f32),
            jax.ShapeDtypeStruct((N, 16), _f32),
            jax.ShapeDtypeStruct((N, 16), _f32),
            jax.ShapeDtypeStruct((1, 16), _f32),
        ],
    )(x, w1, b1r, wc1, was, wad)


def _tc2_body(o0_r, o1_r, den_r, bb_r, rexp_r, wc_r, was_r, wad_r,
              g_r, as_r, ad_r, m_r):
    den8 = den_r[...][:, 0:8]
    dx = jnp.dot(den8, rexp_r[...], preferred_element_type=_f32,
                 precision=lax.Precision.HIGHEST)
    ocat = jnp.concatenate([o0_r[...], o1_r[...]], axis=1)
    h2 = jnp.maximum(ocat / (dx + 1e-16) + bb_r[...], 0.0)
    g = jnp.dot(h2, wc_r[...], preferred_element_type=_f32)
    a_s = jnp.dot(g, was_r[...], preferred_element_type=_f32,
                  precision=lax.Precision.HIGHEST)
    a_d = jnp.dot(g, wad_r[...], preferred_element_type=_f32,
                  precision=lax.Precision.HIGHEST)
    g_r[...] = g
    as_r[...] = a_s
    ad_r[...] = a_d

    @pl.when(pl.program_id(0) == 0)
    def _():
        m_r[...] = jnp.full((1, 16), -jnp.inf, _f32)

    m_r[...] = jnp.maximum(m_r[...], jnp.max(a_s, axis=0, keepdims=True))


def _tc2(o0, o1, den, bbr, rexp, wc, was, wad):
    grid = N // _BM
    return pl.pallas_call(
        _tc2_body,
        grid=(grid,),
        in_specs=[
            pl.BlockSpec((_BM, 128), lambda i: (i, 0)),
            pl.BlockSpec((_BM, 128), lambda i: (i, 0)),
            pl.BlockSpec((_BM, 16), lambda i: (i, 0)),
            pl.BlockSpec((1, D), lambda i: (0, 0)),
            pl.BlockSpec((8, D), lambda i: (0, 0)),
            pl.BlockSpec((D, D), lambda i: (0, 0)),
            pl.BlockSpec((D, 16), lambda i: (0, 0)),
            pl.BlockSpec((D, 16), lambda i: (0, 0)),
        ],
        out_specs=[
            pl.BlockSpec((_BM, D), lambda i: (i, 0)),
            pl.BlockSpec((_BM, 16), lambda i: (i, 0)),
            pl.BlockSpec((_BM, 16), lambda i: (i, 0)),
            pl.BlockSpec((1, 16), lambda i: (0, 0)),
        ],
        out_shape=[
            jax.ShapeDtypeStruct((N, D), _f32),
            jax.ShapeDtypeStruct((N, 16), _f32),
            jax.ShapeDtypeStruct((N, 16), _f32),
            jax.ShapeDtypeStruct((1, 16), _f32),
        ],
    )(o0, o1, den, bbr, rexp, wc, was, wad)


def _tc3_body(o0_r, o1_r, den_r, bb_r, rexp_r, w2_r, b2_r, wcls_r, bcls_r, out_r):
    den8 = den_r[...][:, 0:8]
    dx = jnp.dot(den8, rexp_r[...], preferred_element_type=_f32,
                 precision=lax.Precision.HIGHEST)
    ocat = jnp.concatenate([o0_r[...], o1_r[...]], axis=1)
    o2 = ocat / (dx + 1e-16) + bb_r[...]
    h3 = jnp.maximum(
        jnp.dot(o2, w2_r[...], preferred_element_type=_f32) + b2_r[...], 0.0)
    out_r[...] = jnp.dot(h3, wcls_r[...], preferred_element_type=_f32) + bcls_r[...]


def _tc3(o0, o1, den, bbr, rexp, w2, b2r, wclsp, bclsp):
    return pl.pallas_call(
        _tc3_body,
        grid=(1,),
        in_specs=[
            pl.BlockSpec((1024, 128), lambda i: (0, 0)),
            pl.BlockSpec((1024, 128), lambda i: (0, 0)),
            pl.BlockSpec((1024, 16), lambda i: (0, 0)),
            pl.BlockSpec((1, D), lambda i: (0, 0)),
            pl.BlockSpec((8, D), lambda i: (0, 0)),
            pl.BlockSpec((D, 64), lambda i: (0, 0)),
            pl.BlockSpec((1, 64), lambda i: (0, 0)),
            pl.BlockSpec((64, 128), lambda i: (0, 0)),
            pl.BlockSpec((1, 128), lambda i: (0, 0)),
        ],
        out_specs=[pl.BlockSpec((1024, 128), lambda i: (0, 0))],
        out_shape=[jax.ShapeDtypeStruct((1024, 128), _f32)],
    )(o0, o1, den, bbr, rexp, w2, b2r, wclsp, bclsp)[0]


_sc_mesh = plsc.VectorSubcoreMesh(core_axis_name="c", subcore_axis_name="s")

@functools.partial(
    pl.kernel,
    out_type=(
        jax.ShapeDtypeStruct((2 * NPAD, 128), _f32),
        jax.ShapeDtypeStruct((2 * NPAD, 16), _f32),
    ),
    mesh=_sc_mesh,
    compiler_params=pltpu.CompilerParams(use_tc_tiling_on_sc=False),
    scratch_types=(
        [pltpu.VMEM((B,), jnp.int32)] * 4
        + [pltpu.VMEM((B, 128), _f32),
           pltpu.VMEM((B, 128), _f32),
           pltpu.VMEM((B, 16), _f32),
           pltpu.VMEM((B, 16), _f32),
           pltpu.VMEM((B, 16), _f32)]
        + [pltpu.VMEM((B,), jnp.int32)] * 4
        + [pltpu.VMEM((B, 128), _f32),
           pltpu.VMEM((B, 128), _f32),
           pltpu.VMEM((B, 16), _f32),
           pltpu.VMEM((B, 16), _f32),
           pltpu.VMEM((B, 16), _f32)]
        + [pltpu.VMEM((16,), _f32),
           pltpu.VMEM_SHARED((NPAD, 128), _f32),
           pltpu.VMEM_SHARED((NPAD, 16), _f32)]
        + [pltpu.SemaphoreType.DMA] * 6
    ),
)
def _sc_conv(src_h, dst_h, t_h, as_h, ad_h, m_h, zz_h, zd_h,
             out_h, den_h,
             su0, sg0, dg0, ds0, rows0, rs0, asv0, adv0, wv0,
             su1, sg1, dg1, ds1, rows1, rs1, asv1, adv1, wv1,
             mv, accum, densh, semi0, semi1, semg0, semg1, sems0, sems1):
    cid = lax.axis_index("c")
    tid = lax.axis_index("s")

    pltpu.sync_copy(zz_h, accum.at[pl.ds(tid * ZSLAB, ZSLAB)])
    pltpu.sync_copy(zd_h, densh.at[pl.ds(tid * ZSLAB, ZSLAB)])
    pltpu.sync_copy(m_h, mv)
    plsc.subcore_barrier()

    mm = mv[...]
    off = cid * NPAD
    ebase = tid * (CPT * B)

    bufs = [
        dict(su=su0, sg=sg0, dg=dg0, dsc=ds0, rows=rows0, rs=rs0, asv=asv0,
             adv=adv0, wv=wv0, semi=semi0, semg=semg0, sems=sems0),
        dict(su=su1, sg=sg1, dg=dg1, dsc=ds1, rows=rows1, rs=rs1, asv=asv1,
             adv=adv1, wv=wv1, semi=semi1, semg=semg1, sems=sems1),
    ]

    def issue_idx(i, bp):
        pltpu.async_copy(src_h.at[pl.ds(ebase + i * B, B)], bp["su"], bp["semi"])
        pltpu.async_copy(dst_h.at[pl.ds(ebase + i * B, B)], bp["dg"], bp["semi"])

    def wait_idx(i, bp):
        pltpu.make_async_copy(src_h.at[pl.ds(ebase + i * B, B)], bp["su"],
                              bp["semi"]).wait()
        pltpu.make_async_copy(dst_h.at[pl.ds(ebase + i * B, B)], bp["dg"],
                              bp["semi"]).wait()

    def issue_gathers(bp):
        for j in range(B // 16):
            bp["sg"][pl.ds(j * 16, 16)] = bp["su"][pl.ds(j * 16, 16)] + off
        pltpu.async_copy(as_h.at[bp["su"]], bp["asv"], bp["semg"])
        pltpu.async_copy(ad_h.at[bp["dg"]], bp["adv"], bp["semg"])
        pltpu.async_copy(t_h.at[bp["sg"]], bp["rows"], bp["semg"])

    def wait_gathers(bp):
        pltpu.make_async_copy(as_h.at[bp["su"]], bp["asv"], bp["semg"]).wait()
        pltpu.make_async_copy(ad_h.at[bp["dg"]], bp["adv"], bp["semg"]).wait()
        pltpu.make_async_copy(t_h.at[bp["sg"]], bp["rows"], bp["semg"]).wait()

    def compute(bp, denc):
        rows, rs = bp["rows"], bp["rs"]
        wv, asv, adv = bp["wv"], bp["asv"], bp["adv"]

        def body(ho, store_w):
            def edge(j, c2):
                a = asv[j, :]
                dd = adv[j, :]
                t = a + dd
                alpha = jnp.maximum(t, 0.2 * t)
                u = mm + dd
                cc = jnp.maximum(u, 0.2 * u)
                w = jnp.exp(alpha - cc)
                if store_w:
                    wv[j, :] = w
                for k in range(4):
                    wk = w[ho + k]
                    rs[j, pl.ds(k * 32, 16)] = rows[j, pl.ds(k * 32, 16)] * wk
                    rs[j, pl.ds(k * 32 + 16, 16)] = (
                        rows[j, pl.ds(k * 32 + 16, 16)] * wk)
                return c2

            lax.fori_loop(0, B, edge, 0)

        @pl.when(cid == 0)
        def _():
            body(0, denc == 0)

        @pl.when(cid == 1)
        def _():
            body(4, denc == 1)

    def issue_scatter(bp, denc):
        for j in range(B // 16):
            bp["dsc"][pl.ds(j * 16, 16)] = bp["dg"][pl.ds(j * 16, 16)]
        pltpu.async_copy(bp["rs"], accum.at[bp["dsc"]], bp["sems"], add=True)

        @pl.when(cid == denc)
        def _():
            pltpu.async_copy(bp["wv"], densh.at[bp["dsc"]], bp["sems"], add=True)

    def wait_scatter(bp, denc):
        pltpu.make_async_copy(bp["rs"], accum.at[bp["dsc"]], bp["sems"]).wait()

        @pl.when(cid == denc)
        def _():
            pltpu.make_async_copy(bp["wv"], densh.at[bp["dsc"]], bp["sems"]).wait()

    issue_idx(0, bufs[0])
    wait_idx(0, bufs[0])
    issue_gathers(bufs[0])

    def outer(o, carry):
        for p in (0, 1):
            i = o * 2 + p
            bp = bufs[p]
            bq = bufs[1 - p]

            @pl.when(i + 1 < CPT)
            def _():
                issue_idx(i + 1, bq)

            wait_gathers(bp)

            @pl.when(i + 1 < CPT)
            def _():
                wait_idx(i + 1, bq)
                issue_gathers(bq)

            @pl.when(i >= 2)
            def _():
                wait_scatter(bp, p)

            compute(bp, p)
            issue_scatter(bp, p)
        return carry

    lax.fori_loop(0, CPT // 2, outer, 0)
    wait_scatter(bufs[0], 0)
    wait_scatter(bufs[1], 1)
    plsc.subcore_barrier()

    r0 = tid * ZSLAB
    pltpu.sync_copy(accum.at[pl.ds(r0, ZSLAB)],
                    out_h.at[pl.ds(cid * NPAD + r0, ZSLAB)])

    pltpu.sync_copy(densh.at[pl.ds(r0, ZSLAB)],
                    den_h.at[pl.ds(cid * NPAD + r0, ZSLAB)])


def _att_w(a):
    v = a.reshape(-1)
    w = jnp.zeros((D, HEADS), _f32).at[jnp.arange(D), jnp.arange(D) // CH].set(v)
    return jnp.concatenate([w, w], axis=1)


def _tables(g, a_s, a_d):
    gp = jnp.pad(g, ((0, NPAD - N), (0, 0)))
    t = jnp.concatenate([gp[:, :128], gp[:, 128:]], axis=0)
    asp = jnp.pad(a_s, ((0, NPAD - N), (0, 0)))
    adp = jnp.pad(a_d, ((0, NPAD - N), (0, 0)))
    return t, asp, adp


def kernel(x, W1, b1, Wc1, as1, ad1, bb1, Wc2, as2, ad2, bb2, W2, b2,
           Wcls, bcls, edge_index, batch_size):
    loop = jnp.arange(N, dtype=jnp.int32)
    padi = jnp.full((EP - EE,), DUMMY, jnp.int32)
    src = jnp.concatenate([edge_index[0].astype(jnp.int32), loop, padi])
    dst = jnp.concatenate([edge_index[1].astype(jnp.int32), loop, padi])

    b1r = b1.reshape(1, D)
    bb1r = bb1.reshape(1, D)
    bb2r = bb2.reshape(1, D)
    b2r = b2.reshape(1, 64)
    rexp = jnp.repeat(jnp.eye(HEADS, dtype=_f32), CH, axis=1)
    wclsp = jnp.pad(Wcls, ((0, 0), (0, 126)))
    bclsp = jnp.pad(bcls, (0, 126)).reshape(1, 128)
    was1 = _att_w(as1)
    wad1 = _att_w(ad1)
    was2 = _att_w(as2)
    wad2 = _att_w(ad2)
    zz = jnp.zeros((ZSLAB, 128), _f32)
    zd = jnp.zeros((ZSLAB, 16), _f32)

    g1, s1, d1, m1 = _tc1(x, W1, b1r, Wc1, was1, wad1)
    t1, asp1, adp1 = _tables(g1, s1, d1)
    out1, den1 = _sc_conv(src, dst, t1, asp1, adp1, m1.reshape(16), zz, zd)

    den1s = den1[:N] + den1[NPAD:NPAD + N]
    g2, s2, d2, m2 = _tc2(out1[:N], out1[NPAD:NPAD + N], den1s, bb1r, rexp,
                          Wc2, was2, wad2)
    t2, asp2, adp2 = _tables(g2, s2, d2)
    out2, den2 = _sc_conv(src, dst, t2, asp2, adp2, m2.reshape(16), zz, zd)

    start = batch_size - 1024
    o20 = lax.dynamic_slice(out2, (start, 0), (1024, 128))
    o21 = lax.dynamic_slice(out2, (start + NPAD, 0), (1024, 128))
    den2sum = den2[:N] + den2[NPAD:NPAD + N]
    den2s = lax.dynamic_slice(den2sum, (start, 0), (1024, 16))

    lg = _tc3(o20, o21, den2s, bb2r, rexp, W2, b2r, wclsp, bclsp)
    return lg[:, :2]


# speedup vs baseline: 1.6338x; 1.6338x over previous
import functools

import jax
import jax.numpy as jnp
from jax import lax
from jax.experimental import pallas as pl
from jax.experimental.pallas import tpu as pltpu
from jax.experimental.pallas import tpu_sc as plsc

N = 10000
E = 320000
HEADS = 8
CH = 32
D = HEADS * CH

NC = 2
NS = 16

NPAD = 10112
DUMMY = N
EE = E + N
B = 96
CPT = 216
EP = NS * CPT * B
ZSLAB = NPAD // NS

_f32 = jnp.float32


_BM = 400


def _tc1_body(x_r, w1_r, b1_r, wc1_r, was_r, wad_r, g_r, as_r, ad_r, m_r):
    h1 = jnp.maximum(
        jnp.dot(x_r[...], w1_r[...], preferred_element_type=_f32) + b1_r[...], 0.0)
    g = jnp.dot(h1, wc1_r[...], preferred_element_type=_f32)
    a_s = jnp.dot(g, was_r[...], preferred_element_type=_f32,
                  precision=lax.Precision.HIGHEST)
    a_d = jnp.dot(g, wad_r[...], preferred_element_type=_f32,
                  precision=lax.Precision.HIGHEST)
    g_r[...] = g
    as_r[...] = a_s
    ad_r[...] = a_d

    @pl.when(pl.program_id(0) == 0)
    def _():
        m_r[...] = jnp.full((1, 16), -jnp.inf, _f32)

    m_r[...] = jnp.maximum(m_r[...], jnp.max(a_s, axis=0, keepdims=True))


def _tc1(x, w1, b1r, wc1, was, wad):
    grid = N // _BM
    return pl.pallas_call(
        _tc1_body,
        grid=(grid,),
        in_specs=[
            pl.BlockSpec((_BM, 128), lambda i: (i, 0)),
            pl.BlockSpec((128, D), lambda i: (0, 0)),
            pl.BlockSpec((1, D), lambda i: (0, 0)),
            pl.BlockSpec((D, D), lambda i: (0, 0)),
            pl.BlockSpec((D, 16), lambda i: (0, 0)),
            pl.BlockSpec((D, 16), lambda i: (0, 0)),
        ],
        out_specs=[
            pl.BlockSpec((_BM, D), lambda i: (i, 0)),
            pl.BlockSpec((_BM, 16), lambda i: (i, 0)),
            pl.BlockSpec((_BM, 16), lambda i: (i, 0)),
            pl.BlockSpec((1, 16), lambda i: (0, 0)),
        ],
        out_shape=[
            jax.ShapeDtypeStruct((N, D), _f32),
            jax.ShapeDtypeStruct((N, 16), _f32),
            jax.ShapeDtypeStruct((N, 16), _f32),
            jax.ShapeDtypeStruct((1, 16), _f32),
        ],
    )(x, w1, b1r, wc1, was, wad)


def _tc2_body(o0_r, o1_r, den_r, bb_r, rexp_r, wc_r, was_r, wad_r,
              g_r, as_r, ad_r, m_r):
    den8 = den_r[...][:, 0:8]
    dx = jnp.dot(den8, rexp_r[...], preferred_element_type=_f32,
                 precision=lax.Precision.HIGHEST)
    ocat = jnp.concatenate([o0_r[...], o1_r[...]], axis=1)
    h2 = jnp.maximum(ocat / (dx + 1e-16) + bb_r[...], 0.0)
    g = jnp.dot(h2, wc_r[...], preferred_element_type=_f32)
    a_s = jnp.dot(g, was_r[...], preferred_element_type=_f32,
                  precision=lax.Precision.HIGHEST)
    a_d = jnp.dot(g, wad_r[...], preferred_element_type=_f32,
                  precision=lax.Precision.HIGHEST)
    g_r[...] = g
    as_r[...] = a_s
    ad_r[...] = a_d

    @pl.when(pl.program_id(0) == 0)
    def _():
        m_r[...] = jnp.full((1, 16), -jnp.inf, _f32)

    m_r[...] = jnp.maximum(m_r[...], jnp.max(a_s, axis=0, keepdims=True))


def _tc2(o0, o1, den, bbr, rexp, wc, was, wad):
    grid = N // _BM
    return pl.pallas_call(
        _tc2_body,
        grid=(grid,),
        in_specs=[
            pl.BlockSpec((_BM, 128), lambda i: (i, 0)),
            pl.BlockSpec((_BM, 128), lambda i: (i, 0)),
            pl.BlockSpec((_BM, 16), lambda i: (i, 0)),
            pl.BlockSpec((1, D), lambda i: (0, 0)),
            pl.BlockSpec((8, D), lambda i: (0, 0)),
            pl.BlockSpec((D, D), lambda i: (0, 0)),
            pl.BlockSpec((D, 16), lambda i: (0, 0)),
            pl.BlockSpec((D, 16), lambda i: (0, 0)),
        ],
        out_specs=[
            pl.BlockSpec((_BM, D), lambda i: (i, 0)),
            pl.BlockSpec((_BM, 16), lambda i: (i, 0)),
            pl.BlockSpec((_BM, 16), lambda i: (i, 0)),
            pl.BlockSpec((1, 16), lambda i: (0, 0)),
        ],
        out_shape=[
            jax.ShapeDtypeStruct((N, D), _f32),
            jax.ShapeDtypeStruct((N, 16), _f32),
            jax.ShapeDtypeStruct((N, 16), _f32),
            jax.ShapeDtypeStruct((1, 16), _f32),
        ],
    )(o0, o1, den, bbr, rexp, wc, was, wad)


def _tc3_body(o0_r, o1_r, den_r, bb_r, rexp_r, w2_r, b2_r, wcls_r, bcls_r, out_r):
    den8 = den_r[...][:, 0:8]
    dx = jnp.dot(den8, rexp_r[...], preferred_element_type=_f32,
                 precision=lax.Precision.HIGHEST)
    ocat = jnp.concatenate([o0_r[...], o1_r[...]], axis=1)
    o2 = ocat / (dx + 1e-16) + bb_r[...]
    h3 = jnp.maximum(
        jnp.dot(o2, w2_r[...], preferred_element_type=_f32) + b2_r[...], 0.0)
    out_r[...] = jnp.dot(h3, wcls_r[...], preferred_element_type=_f32) + bcls_r[...]


def _tc3(o0, o1, den, bbr, rexp, w2, b2r, wclsp, bclsp):
    return pl.pallas_call(
        _tc3_body,
        grid=(1,),
        in_specs=[
            pl.BlockSpec((1024, 128), lambda i: (0, 0)),
            pl.BlockSpec((1024, 128), lambda i: (0, 0)),
            pl.BlockSpec((1024, 16), lambda i: (0, 0)),
            pl.BlockSpec((1, D), lambda i: (0, 0)),
            pl.BlockSpec((8, D), lambda i: (0, 0)),
            pl.BlockSpec((D, 64), lambda i: (0, 0)),
            pl.BlockSpec((1, 64), lambda i: (0, 0)),
            pl.BlockSpec((64, 128), lambda i: (0, 0)),
            pl.BlockSpec((1, 128), lambda i: (0, 0)),
        ],
        out_specs=[pl.BlockSpec((1024, 128), lambda i: (0, 0))],
        out_shape=[jax.ShapeDtypeStruct((1024, 128), _f32)],
    )(o0, o1, den, bbr, rexp, w2, b2r, wclsp, bclsp)[0]


_sc_mesh = plsc.VectorSubcoreMesh(core_axis_name="c", subcore_axis_name="s")

@functools.partial(
    pl.kernel,
    out_type=(
        jax.ShapeDtypeStruct((2 * NPAD, 128), _f32),
        jax.ShapeDtypeStruct((2 * NPAD, 16), _f32),
    ),
    mesh=_sc_mesh,
    compiler_params=pltpu.CompilerParams(use_tc_tiling_on_sc=False),
    scratch_types=(
        [pltpu.VMEM((B,), jnp.int32)] * 4
        + [pltpu.VMEM((B, 128), _f32),
           pltpu.VMEM((B, 16), _f32),
           pltpu.VMEM((B, 16), _f32),
           pltpu.VMEM((B, 16), _f32)]
        + [pltpu.VMEM((B,), jnp.int32)] * 4
        + [pltpu.VMEM((B, 128), _f32),
           pltpu.VMEM((B, 16), _f32),
           pltpu.VMEM((B, 16), _f32),
           pltpu.VMEM((B, 16), _f32)]
        + [pltpu.VMEM((16,), _f32),
           pltpu.VMEM_SHARED((NPAD, 128), _f32),
           pltpu.VMEM_SHARED((NPAD, 16), _f32)]
        + [pltpu.SemaphoreType.DMA] * 6
    ),
)
def _sc_conv(src_h, dst_h, t_h, as_h, ad_h, m_h, zz_h, zd_h,
             out_h, den_h,
             su0, sg0, dg0, ds0, rows0, asv0, adv0, wv0,
             su1, sg1, dg1, ds1, rows1, asv1, adv1, wv1,
             mv, accum, densh, semi0, semi1, semg0, semg1, sems0, sems1):
    cid = lax.axis_index("c")
    tid = lax.axis_index("s")

    pltpu.sync_copy(zz_h, accum.at[pl.ds(tid * ZSLAB, ZSLAB)])
    pltpu.sync_copy(zd_h, densh.at[pl.ds(tid * ZSLAB, ZSLAB)])
    pltpu.sync_copy(m_h, mv)
    plsc.subcore_barrier()

    mm = mv[...]
    off = cid * NPAD
    ebase = tid * (CPT * B)

    bufs = [
        dict(su=su0, sg=sg0, dg=dg0, dsc=ds0, rows=rows0, asv=asv0, adv=adv0,
             wv=wv0, semi=semi0, semg=semg0, sems=sems0),
        dict(su=su1, sg=sg1, dg=dg1, dsc=ds1, rows=rows1, asv=asv1, adv=adv1,
             wv=wv1, semi=semi1, semg=semg1, sems=sems1),
    ]

    def issue_idx(i, bp):
        pltpu.async_copy(src_h.at[pl.ds(ebase + i * B, B)], bp["su"], bp["semi"])
        pltpu.async_copy(dst_h.at[pl.ds(ebase + i * B, B)], bp["dg"], bp["semi"])

    def wait_idx(i, bp):
        pltpu.make_async_copy(src_h.at[pl.ds(ebase + i * B, B)], bp["su"],
                              bp["semi"]).wait()
        pltpu.make_async_copy(dst_h.at[pl.ds(ebase + i * B, B)], bp["dg"],
                              bp["semi"]).wait()

    def issue_gathers(bp):
        for j in range(B // 16):
            bp["sg"][pl.ds(j * 16, 16)] = bp["su"][pl.ds(j * 16, 16)] + off
        pltpu.async_copy(as_h.at[bp["su"]], bp["asv"], bp["semg"])
        pltpu.async_copy(ad_h.at[bp["dg"]], bp["adv"], bp["semg"])
        pltpu.async_copy(t_h.at[bp["sg"]], bp["rows"], bp["semg"])

    def wait_gathers(bp):
        pltpu.make_async_copy(as_h.at[bp["su"]], bp["asv"], bp["semg"]).wait()
        pltpu.make_async_copy(ad_h.at[bp["dg"]], bp["adv"], bp["semg"]).wait()
        pltpu.make_async_copy(t_h.at[bp["sg"]], bp["rows"], bp["semg"]).wait()

    def compute(bp, denc):
        rows, wv, asv, adv = bp["rows"], bp["wv"], bp["asv"], bp["adv"]

        def body(ho, store_w):
            def edge(j, c2):
                a = asv[j, :]
                dd = adv[j, :]
                t = a + dd
                alpha = jnp.maximum(t, 0.2 * t)
                u = mm + dd
                cc = jnp.maximum(u, 0.2 * u)
                w = jnp.exp(alpha - cc)
                if store_w:
                    wv[j, :] = w
                for k in range(4):
                    wk = w[ho + k]
                    rows[j, pl.ds(k * 32, 16)] = rows[j, pl.ds(k * 32, 16)] * wk
                    rows[j, pl.ds(k * 32 + 16, 16)] = (
                        rows[j, pl.ds(k * 32 + 16, 16)] * wk)
                return c2

            lax.fori_loop(0, B, edge, 0)

        @pl.when(cid == 0)
        def _():
            body(0, denc == 0)

        @pl.when(cid == 1)
        def _():
            body(4, denc == 1)

    def issue_scatter(bp, denc):
        for j in range(B // 16):
            bp["dsc"][pl.ds(j * 16, 16)] = bp["dg"][pl.ds(j * 16, 16)]
        pltpu.async_copy(bp["rows"], accum.at[bp["dsc"]], bp["sems"], add=True)

        @pl.when(cid == denc)
        def _():
            pltpu.async_copy(bp["wv"], densh.at[bp["dsc"]], bp["sems"], add=True)

    def wait_scatter(bp, denc):
        pltpu.make_async_copy(bp["rows"], accum.at[bp["dsc"]], bp["sems"]).wait()

        @pl.when(cid == denc)
        def _():
            pltpu.make_async_copy(bp["wv"], densh.at[bp["dsc"]], bp["sems"]).wait()

    issue_idx(0, bufs[0])
    wait_idx(0, bufs[0])
    issue_gathers(bufs[0])

    def outer(o, carry):
        for p in (0, 1):
            i = o * 2 + p
            bp = bufs[p]
            bq = bufs[1 - p]

            @pl.when(i + 1 < CPT)
            def _():
                issue_idx(i + 1, bq)

            wait_gathers(bp)

            @pl.when(i >= 1)
            def _():
                wait_scatter(bq, 1 - p)

            @pl.when(i + 1 < CPT)
            def _():
                wait_idx(i + 1, bq)
                issue_gathers(bq)

            compute(bp, p)
            issue_scatter(bp, p)
        return carry

    lax.fori_loop(0, CPT // 2, outer, 0)
    wait_scatter(bufs[(CPT - 1) % 2], (CPT - 1) % 2)
    plsc.subcore_barrier()

    r0 = tid * ZSLAB
    pltpu.sync_copy(accum.at[pl.ds(r0, ZSLAB)],
                    out_h.at[pl.ds(cid * NPAD + r0, ZSLAB)])

    pltpu.sync_copy(densh.at[pl.ds(r0, ZSLAB)],
                    den_h.at[pl.ds(cid * NPAD + r0, ZSLAB)])


def _att_w(a):
    v = a.reshape(-1)
    w = jnp.zeros((D, HEADS), _f32).at[jnp.arange(D), jnp.arange(D) // CH].set(v)
    return jnp.concatenate([w, w], axis=1)


def _tables(g, a_s, a_d):
    gp = jnp.pad(g, ((0, NPAD - N), (0, 0)))
    t = jnp.concatenate([gp[:, :128], gp[:, 128:]], axis=0)
    asp = jnp.pad(a_s, ((0, NPAD - N), (0, 0)))
    adp = jnp.pad(a_d, ((0, NPAD - N), (0, 0)))
    return t, asp, adp


def kernel(x, W1, b1, Wc1, as1, ad1, bb1, Wc2, as2, ad2, bb2, W2, b2,
           Wcls, bcls, edge_index, batch_size):
    loop = jnp.arange(N, dtype=jnp.int32)
    padi = jnp.full((EP - EE,), DUMMY, jnp.int32)
    src = jnp.concatenate([edge_index[0].astype(jnp.int32), loop, padi])
    dst = jnp.concatenate([edge_index[1].astype(jnp.int32), loop, padi])

    b1r = b1.reshape(1, D)
    bb1r = bb1.reshape(1, D)
    bb2r = bb2.reshape(1, D)
    b2r = b2.reshape(1, 64)
    rexp = jnp.repeat(jnp.eye(HEADS, dtype=_f32), CH, axis=1)
    wclsp = jnp.pad(Wcls, ((0, 0), (0, 126)))
    bclsp = jnp.pad(bcls, (0, 126)).reshape(1, 128)
    was1 = _att_w(as1)
    wad1 = _att_w(ad1)
    was2 = _att_w(as2)
    wad2 = _att_w(ad2)
    zz = jnp.zeros((ZSLAB, 128), _f32)
    zd = jnp.zeros((ZSLAB, 16), _f32)

    g1, s1, d1, m1 = _tc1(x, W1, b1r, Wc1, was1, wad1)
    t1, asp1, adp1 = _tables(g1, s1, d1)
    out1, den1 = _sc_conv(src, dst, t1, asp1, adp1, m1.reshape(16), zz, zd)

    den1s = den1[:N] + den1[NPAD:NPAD + N]
    g2, s2, d2, m2 = _tc2(out1[:N], out1[NPAD:NPAD + N], den1s, bb1r, rexp,
                          Wc2, was2, wad2)
    t2, asp2, adp2 = _tables(g2, s2, d2)
    out2, den2 = _sc_conv(src, dst, t2, asp2, adp2, m2.reshape(16), zz, zd)

    start = batch_size - 1024
    o20 = lax.dynamic_slice(out2, (start, 0), (1024, 128))
    o21 = lax.dynamic_slice(out2, (start + NPAD, 0), (1024, 128))
    den2sum = den2[:N] + den2[NPAD:NPAD + N]
    den2s = lax.dynamic_slice(den2sum, (start, 0), (1024, 16))

    lg = _tc3(o20, o21, den2s, bb2r, rexp, W2, b2r, wclsp, bclsp)
    return lg[:, :2]
